# Initial kernel scaffold; baseline (speedup 1.0000x reference)
import jax
import jax.numpy as jnp
from jax.experimental import pallas as pl


def kernel(inputs, embedding_weight):
    raise NotImplementedError("write your pallas kernel here")


# speedup vs baseline: 1.2877x; 1.2877x over previous
import jax
import jax.numpy as jnp
from jax.experimental import pallas as pl
from jax.experimental.pallas import tpu as pltpu
from jax.experimental.pallas import tpu_sc as plsc

_DIM = 256
_NUM_EMB = 8192
_TM = 1024
_NCHUNK = 1024
_COMMIT = 0.25
_GATHER_WINDOW = 128


def _argmin_body(x2_ref, xn_ref, en_ref, e_ref, idx_ref, dsum_ref):
    x2 = x2_ref[...]
    xn = xn_ref[...]
    run_min = jnp.full((_TM, 1), jnp.inf, jnp.float32)
    run_idx = jnp.full((_TM, 1), jnp.float32(3e38))
    col = jax.lax.broadcasted_iota(jnp.int32, (_TM, _NCHUNK), 1).astype(jnp.float32)
    _BOUNDS = (2731, 5462)
    for c in range(_NUM_EMB // _NCHUNK):
        e_c = e_ref[pl.ds(c * _NCHUNK, _NCHUNK), :]
        en_c = en_ref[:, pl.ds(c * _NCHUNK, _NCHUNK)]
        m2 = jax.lax.dot_general(
            x2, e_c, (((1,), (1,)), ((), ())),
            preferred_element_type=jnp.float32,
        )
        d = (xn + en_c) - m2
        gcol = col + jnp.float32(c * _NCHUNK)
        lo = c * _NCHUNK
        hi = lo + _NCHUNK
        cuts = [b for b in _BOUNDS if lo < b < hi]
        segs = [lo] + cuts + [hi]
        for s0, s1 in zip(segs[:-1], segs[1:]):
            if (s0, s1) == (lo, hi):
                ds = d
            else:
                inseg = (gcol >= jnp.float32(s0)) & (gcol < jnp.float32(s1))
                ds = jnp.where(inseg, d, jnp.float32(jnp.inf))
            c_min = jnp.min(ds, axis=1, keepdims=True)
            c_idx = jnp.min(
                jnp.where(ds == c_min, gcol, jnp.float32(3e38)),
                axis=1, keepdims=True,
            )
            better = c_min < run_min
            run_idx = jnp.where(better, c_idx, run_idx)
            run_min = jnp.where(better, c_min, run_min)
            if s1 in _BOUNDS:
                run_min = run_min.astype(jnp.bfloat16).astype(jnp.float32)
    idx_ref[...] = run_idx.astype(jnp.int32)
    dsum_ref[0, 0, 0] = jnp.sum(run_min)


def _tc_argmin(flat, xnorm, enorm_row, weight):
    tokens = flat.shape[0]
    return pl.pallas_call(
        _argmin_body,
        grid=(tokens // _TM,),
        in_specs=[
            pl.BlockSpec((_TM, _DIM), lambda m: (m, 0)),
            pl.BlockSpec((_TM, 1), lambda m: (m, 0)),
            pl.BlockSpec((1, _NUM_EMB), lambda m: (0, 0)),
            pl.BlockSpec((_NUM_EMB, _DIM), lambda m: (0, 0)),
        ],
        out_specs=[
            pl.BlockSpec((_TM, 1), lambda m: (m, 0)),
            pl.BlockSpec((1, 1, 1), lambda m: (m, 0, 0), memory_space=pltpu.SMEM),
        ],
        out_shape=[
            jax.ShapeDtypeStruct((tokens, 1), jnp.int32),
            jax.ShapeDtypeStruct((tokens // _TM, 1, 1), jnp.float32),
        ],
        compiler_params=pltpu.CompilerParams(
            dimension_semantics=("parallel",),
        ),
    )(flat, xnorm, enorm_row, weight)


def _sc_gather(weight, idx_row):
    tokens = idx_row.shape[1]
    mesh = plsc.VectorSubcoreMesh(core_axis_name="c", subcore_axis_name="s")

    @pl.kernel(
        out_type=jax.ShapeDtypeStruct((tokens, _DIM), jnp.float32),
        mesh=mesh,
    )
    def k(w_hbm, i_hbm, o_hbm):
        def body(i_vmem, o_vmem):
            pltpu.sync_copy(w_hbm.at[i_vmem.at[0]], o_vmem)

        pltpu.emit_pipeline(
            body,
            grid=(tokens // _GATHER_WINDOW,),
            in_specs=[pl.BlockSpec((1, _GATHER_WINDOW), lambda i: (0, i))],
            out_specs=[pl.BlockSpec((_GATHER_WINDOW, _DIM), lambda i: (i, 0))],
            core_axis_name=("c", "s"),
            dimension_semantics=(pltpu.PARALLEL,),
        )(i_hbm, o_hbm)

    return k(weight, idx_row)


def kernel(inputs, embedding_weight):
    inputs_shape = inputs.shape
    flat = inputs.reshape(-1, _DIM)
    tokens = flat.shape[0]
    xnorm = jnp.sum(flat**2, axis=1, keepdims=True)
    enorm = jnp.sum(embedding_weight**2, axis=1)
    idx2d, dsum = _tc_argmin(
        (flat + flat).astype(jnp.bfloat16),
        xnorm,
        enorm.reshape(1, _NUM_EMB),
        embedding_weight.astype(jnp.bfloat16),
    )
    q = _sc_gather(embedding_weight, idx2d.reshape(1, tokens))
    vq_loss = _COMMIT * (jnp.sum(dsum) / jnp.float32(tokens * _DIM))
    quantized_ste = q.reshape(inputs_shape)
    indices = idx2d.reshape(inputs_shape[:-1])
    return (vq_loss, quantized_ste, indices)
